# Initial kernel scaffold; baseline (speedup 1.0000x reference)
import jax
import jax.numpy as jnp
from jax.experimental import pallas as pl


def kernel(x, conv_w, conv_b):
    raise NotImplementedError("write your pallas kernel here")


# speedup vs baseline: 290.4645x; 290.4645x over previous
import functools

import jax
import jax.numpy as jnp
from jax.experimental import pallas as pl

_SAMPLES = 256
_KK = 3
_BIG = 1e30
_NEG = -3e38


def _tc_body(x_ref, xs_ref, rand_ref, w_ref, b_ref, out_ref, *, T):
    xt = x_ref[0]
    xs = xs_ref[0]
    norm = jnp.sqrt(jnp.sum(xs * xs, axis=0, keepdims=True))
    sn = xs / (norm + 1e-8)
    norm_t = jnp.sqrt(jnp.sum(xt * xt, axis=0, keepdims=True))
    xn = xt / (norm_t + 1e-8)
    scores = jax.lax.dot_general(
        xn, sn, (((0,), (0,)), ((), ())))
    row_id = (jax.lax.broadcasted_iota(jnp.int32, (T, _SAMPLES), 0)
              + pl.program_id(1) * T)
    scores = jnp.where(row_id == rand_ref[...], _BIG, scores)
    iota_s = jax.lax.broadcasted_iota(jnp.int32, (T, _SAMPLES), 1)
    acc = jnp.broadcast_to(b_ref[...], (32, T))
    for k in range(_KK):
        m = jnp.max(scores, axis=1, keepdims=True)
        a = jnp.min(jnp.where(scores == m, iota_s, _SAMPLES),
                    axis=1, keepdims=True)
        oh = (iota_s == a).astype(jnp.float32)
        if k < _KK - 1:
            scores = jnp.where(iota_s == a, _NEG, scores)
        wk = w_ref[k]
        projk = jax.lax.dot_general(
            xs, wk, (((0,), (0,)), ((), ())),
            precision=jax.lax.Precision.HIGHEST)
        acc = acc + jax.lax.dot_general(
            projk, oh, (((0,), (1,)), ((), ())),
            precision=jax.lax.Precision.HIGHEST)
    out_ref[0] = acc


def kernel(x, conv_w, conv_b):
    B, C, H, W = x.shape
    N = H * W
    O = conv_w.shape[0]
    T = 3584
    NT = N // T
    x_flat = x.reshape(B, C, N)
    rand_idx = jax.random.permutation(jax.random.key(42), N)[:_SAMPLES]
    x_sample = jnp.take(x_flat, rand_idx, axis=2)
    rand_row = rand_idx.astype(jnp.int32).reshape(1, _SAMPLES)
    w_r = jnp.transpose(conv_w, (2, 1, 0))
    b_r = conv_b.reshape(O, 1)

    out = pl.pallas_call(
        functools.partial(_tc_body, T=T),
        grid=(B, NT),
        in_specs=[
            pl.BlockSpec((1, C, T), lambda b, t: (b, 0, t)),
            pl.BlockSpec((1, C, _SAMPLES), lambda b, t: (b, 0, 0)),
            pl.BlockSpec((1, _SAMPLES), lambda b, t: (0, 0)),
            pl.BlockSpec((_KK, C, O), lambda b, t: (0, 0, 0)),
            pl.BlockSpec((O, 1), lambda b, t: (0, 0)),
        ],
        out_specs=pl.BlockSpec((1, O, T), lambda b, t: (b, 0, t)),
        out_shape=jax.ShapeDtypeStruct((B, O, N), jnp.float32),
    )(x_flat, x_sample, rand_row, w_r, b_r)
    return out.reshape(B, O, H, W)


# speedup vs baseline: 582.6359x; 2.0059x over previous
import functools

import jax
import jax.numpy as jnp
from jax.experimental import pallas as pl

_SAMPLES = 256
_KK = 3
_BIG = 1e30
_NEG = -3e38


def _tc_body(x_ref, xs_ref, rand_ref, w_ref, b_ref, out_ref, *, T):
    xt = x_ref[0]
    xs = xs_ref[0]
    norm = jnp.sqrt(jnp.sum(xs * xs, axis=0, keepdims=True))
    sn = xs / (norm + 1e-8)
    norm_t = jnp.sqrt(jnp.sum(xt * xt, axis=0, keepdims=True))
    xn = xt / (norm_t + 1e-8)
    scores = jax.lax.dot_general(
        xn, sn, (((0,), (0,)), ((), ())))
    row_id = (jax.lax.broadcasted_iota(jnp.int32, (T, _SAMPLES), 0)
              + pl.program_id(1) * T)
    scores = jnp.where(row_id == rand_ref[...], _BIG, scores)
    acc = jnp.broadcast_to(b_ref[...], (32, T))
    for k in range(_KK):
        m = jnp.max(scores, axis=1, keepdims=True)
        oh_b = scores == m
        if k < _KK - 1:
            scores = jnp.where(oh_b, _NEG, scores)
        wk = w_ref[k]
        projk = jax.lax.dot_general(
            xs, wk, (((0,), (0,)), ((), ())),
            precision=jax.lax.Precision.HIGHEST)
        acc = acc + jax.lax.dot_general(
            projk, oh_b.astype(jnp.float32),
            (((0,), (1,)), ((), ())))
    out_ref[0] = acc


def kernel(x, conv_w, conv_b):
    B, C, H, W = x.shape
    N = H * W
    O = conv_w.shape[0]
    T = 3584
    NT = N // T
    x_flat = x.reshape(B, C, N)
    rand_idx = jax.random.permutation(jax.random.key(42), N)[:_SAMPLES]
    x_sample = jnp.take(x_flat, rand_idx, axis=2)
    rand_row = rand_idx.astype(jnp.int32).reshape(1, _SAMPLES)
    w_r = jnp.transpose(conv_w, (2, 1, 0))
    b_r = conv_b.reshape(O, 1)

    out = pl.pallas_call(
        functools.partial(_tc_body, T=T),
        grid=(B, NT),
        in_specs=[
            pl.BlockSpec((1, C, T), lambda b, t: (b, 0, t)),
            pl.BlockSpec((1, C, _SAMPLES), lambda b, t: (b, 0, 0)),
            pl.BlockSpec((1, _SAMPLES), lambda b, t: (0, 0)),
            pl.BlockSpec((_KK, C, O), lambda b, t: (0, 0, 0)),
            pl.BlockSpec((O, 1), lambda b, t: (0, 0)),
        ],
        out_specs=pl.BlockSpec((1, O, T), lambda b, t: (b, 0, t)),
        out_shape=jax.ShapeDtypeStruct((B, O, N), jnp.float32),
    )(x_flat, x_sample, rand_row, w_r, b_r)
    return out.reshape(B, O, H, W)


# speedup vs baseline: 628.4122x; 1.0786x over previous
import functools

import jax
import jax.numpy as jnp
from jax import lax
from jax.experimental import pallas as pl

_S = 256
_KK = 3
_BIG = 1e30
_NEG = -3e38


def _tc_body(x_ref, xs_ref, rand_ref, w_ref, b_ref, out_ref, *, T):
    xt = x_ref[0]
    xs = xs_ref[0]
    norm = jnp.sqrt(jnp.sum(xs * xs, axis=0, keepdims=True))
    sn = xs / (norm + 1e-8)
    norm_t = jnp.sqrt(jnp.sum(xt * xt, axis=0, keepdims=True))
    xn = xt / (norm_t + 1e-8)
    scores = lax.dot_general(sn, xn, (((0,), (0,)), ((), ())))
    tok_id = (lax.broadcasted_iota(jnp.int32, (_S, T), 1)
              + pl.program_id(1) * T)
    scores = jnp.where(tok_id == rand_ref[...], _BIG, scores)
    acc = jnp.broadcast_to(b_ref[...], (32, T))
    for k in range(_KK):
        m = jnp.max(scores, axis=0, keepdims=True)
        oh_b = scores == m
        if k < _KK - 1:
            scores = jnp.where(oh_b, _NEG, scores)
        projk = lax.dot_general(
            xs, w_ref[k], (((0,), (0,)), ((), ())),
            precision=lax.Precision.HIGHEST)
        acc = acc + lax.dot_general(
            projk, oh_b.astype(jnp.float32),
            (((0,), (0,)), ((), ())))
    out_ref[0] = acc


def kernel(x, conv_w, conv_b):
    B, C, H, W = x.shape
    N = H * W
    O = conv_w.shape[0]
    T = 3584
    NT = N // T
    x_flat = x.reshape(B, C, N)
    rand_idx = jax.random.permutation(jax.random.key(42), N)[:_S]
    x_sample = jnp.take(x_flat, rand_idx, axis=2)
    rand_col = rand_idx.astype(jnp.int32).reshape(_S, 1)
    w_r = jnp.transpose(conv_w, (2, 1, 0))
    b_r = conv_b.reshape(O, 1)

    out = pl.pallas_call(
        functools.partial(_tc_body, T=T),
        grid=(B, NT),
        in_specs=[
            pl.BlockSpec((1, C, T), lambda b, t: (b, 0, t)),
            pl.BlockSpec((1, C, _S), lambda b, t: (b, 0, 0)),
            pl.BlockSpec((_S, 1), lambda b, t: (0, 0)),
            pl.BlockSpec((_KK, C, O), lambda b, t: (0, 0, 0)),
            pl.BlockSpec((O, 1), lambda b, t: (0, 0)),
        ],
        out_specs=pl.BlockSpec((1, O, T), lambda b, t: (b, 0, t)),
        out_shape=jax.ShapeDtypeStruct((B, O, N), jnp.float32),
    )(x_flat, x_sample, rand_col, w_r, b_r)
    return out.reshape(B, O, H, W)


# speedup vs baseline: 668.9548x; 1.0645x over previous
import functools

import jax
import jax.numpy as jnp
from jax import lax
from jax.experimental import pallas as pl

_S = 256
_KK = 3
_BIG = 1e30
_NEG = -3e38


def _tc_body(x_ref, xs_ref, rand_ref, w_ref, b_ref, out_ref, *, T):
    xt = x_ref[0]
    xs = xs_ref[0]
    norm = jnp.sqrt(jnp.sum(xs * xs, axis=0, keepdims=True))
    sn = xs / (norm + 1e-8)
    norm_t = jnp.sqrt(jnp.sum(xt * xt, axis=0, keepdims=True))
    xn = xt / (norm_t + 1e-8)
    scores = lax.dot_general(sn, xn, (((0,), (0,)), ((), ())))
    tok_id = (lax.broadcasted_iota(jnp.int32, (_S, T), 1)
              + pl.program_id(1) * T)
    scores = jnp.where(tok_id == rand_ref[...], _BIG, scores)
    acc = jnp.broadcast_to(b_ref[...], (32, T))
    for k in range(_KK):
        m = jnp.max(scores, axis=0, keepdims=True)
        oh_b = scores == m
        if k < _KK - 1:
            scores = jnp.where(oh_b, _NEG, scores)
        projk = lax.dot_general(
            xs, w_ref[k], (((0,), (0,)), ((), ())),
            precision=lax.Precision.HIGHEST)
        acc = acc + lax.dot_general(
            projk, oh_b.astype(jnp.float32),
            (((0,), (0,)), ((), ())))
    out_ref[0] = acc


def kernel(x, conv_w, conv_b):
    B, C, H, W = x.shape
    N = H * W
    O = conv_w.shape[0]
    T = 7168
    NT = N // T
    x_flat = x.reshape(B, C, N)
    rand_idx = jax.random.permutation(jax.random.key(42), N)[:_S]
    x_sample = jnp.take(x_flat, rand_idx, axis=2)
    rand_col = rand_idx.astype(jnp.int32).reshape(_S, 1)
    w_r = jnp.transpose(conv_w, (2, 1, 0))
    b_r = conv_b.reshape(O, 1)

    out = pl.pallas_call(
        functools.partial(_tc_body, T=T),
        grid=(B, NT),
        in_specs=[
            pl.BlockSpec((1, C, T), lambda b, t: (b, 0, t)),
            pl.BlockSpec((1, C, _S), lambda b, t: (b, 0, 0)),
            pl.BlockSpec((_S, 1), lambda b, t: (0, 0)),
            pl.BlockSpec((_KK, C, O), lambda b, t: (0, 0, 0)),
            pl.BlockSpec((O, 1), lambda b, t: (0, 0)),
        ],
        out_specs=pl.BlockSpec((1, O, T), lambda b, t: (b, 0, t)),
        out_shape=jax.ShapeDtypeStruct((B, O, N), jnp.float32),
    )(x_flat, x_sample, rand_col, w_r, b_r)
    return out.reshape(B, O, H, W)
